# Initial kernel scaffold; baseline (speedup 1.0000x reference)
import jax
import jax.numpy as jnp
from jax.experimental import pallas as pl


def kernel(input, edge_index, last_graph_weights, p, Wih, Whh, bih, bhh, Wg, bg):
    raise NotImplementedError("write your pallas kernel here")


# speedup vs baseline: 19.6025x; 19.6025x over previous
import functools

import jax
import jax.numpy as jnp
from jax import lax
from jax.experimental import pallas as pl
from jax.experimental.pallas import tpu as pltpu
from jax.experimental.pallas import tpu_sc as plsc

B, N, C, H, O, K = 4, 10000, 64, 32, 32, 16
E = 1280000
NN = B * N
GH = C * H
GI = K * C


def _proj_body(inp_ref, prow_ref, pcol_ref, y_ref):
    prow = prow_ref[...]
    pn = lax.rsqrt(jnp.sum(prow * prow))
    y = lax.dot_general(inp_ref[...], pcol_ref[...],
                        (((1,), (0,)), ((), ())),
                        preferred_element_type=jnp.float32)
    y_ref[...] = y * pn


def _proj(inp2, prow, pcol):
    bn = 8000
    return pl.pallas_call(
        _proj_body,
        grid=(NN // bn,),
        in_specs=[
            pl.BlockSpec((bn, C), lambda i: (i, 0)),
            pl.BlockSpec((1, C), lambda i: (0, 0)),
            pl.BlockSpec((C, 1), lambda i: (0, 0)),
        ],
        out_specs=pl.BlockSpec((bn, 1), lambda i: (i, 0)),
        out_shape=jax.ShapeDtypeStruct((NN, 1), jnp.float32),
    )(inp2, prow, pcol)


def _topk_body(inp_ref, y_ref, xs_ref):
    y = y_ref[...]
    iota = lax.broadcasted_iota(jnp.int32, (B, N), 1)
    for k in range(K):
        m = jnp.max(y, axis=1, keepdims=True)
        idxv = jnp.min(jnp.where(y == m, iota, N), axis=1, keepdims=True)
        oh = (iota == idxv).astype(jnp.float32)
        for b in range(B):
            row = lax.dot_general(oh[b:b + 1], inp_ref[b],
                                  (((1,), (0,)), ((), ())),
                                  preferred_element_type=jnp.float32)
            xs_ref[b, pl.ds(k, 1), :] = row * jnp.tanh(m[b:b + 1, 0:1])
        y = jnp.where(iota == idxv, -jnp.inf, y)


def _topk(inp, y2):
    return pl.pallas_call(
        _topk_body,
        out_shape=jax.ShapeDtypeStruct((B, K, C), jnp.float32),
    )(inp, y2)


_BJ = 256


def _gru_body(xs_ref, hf_ref, hb_ref, wih_ref, whh_ref, bih_ref, bhh_ref,
              w_ref):
    xs = xs_ref[...]
    hf = hf_ref[...]
    g = []
    for gi in range(3):
        a = lax.dot_general(xs, wih_ref[gi], (((1,), (1,)), ((), ())),
                            preferred_element_type=jnp.float32)
        a = a + bih_ref[gi]
        c = lax.dot_general(hf, whh_ref[gi], (((1,), (1,)), ((), ())),
                            preferred_element_type=jnp.float32)
        c = c + bhh_ref[gi]
        g.append((a, c))
    r = jax.nn.sigmoid(g[0][0] + g[0][1])
    z = jax.nn.sigmoid(g[1][0] + g[1][1])
    n = jnp.tanh(g[2][0] + r * g[2][1])
    w_ref[...] = (1.0 - z) * n + z * hb_ref[...]


def _gru(xs2, hfull, wih3, whh3, bih3, bhh3):
    return pl.pallas_call(
        _gru_body,
        grid=(GH // _BJ,),
        in_specs=[
            pl.BlockSpec((B, GI), lambda j: (0, 0)),
            pl.BlockSpec((B, GH), lambda j: (0, 0)),
            pl.BlockSpec((B, _BJ), lambda j: (0, j)),
            pl.BlockSpec((3, _BJ, GI), lambda j: (0, j, 0)),
            pl.BlockSpec((3, _BJ, GH), lambda j: (0, j, 0)),
            pl.BlockSpec((3, 1, _BJ), lambda j: (0, 0, j)),
            pl.BlockSpec((3, 1, _BJ), lambda j: (0, 0, j)),
        ],
        out_specs=pl.BlockSpec((B, _BJ), lambda j: (0, j)),
        out_shape=jax.ShapeDtypeStruct((B, GH), jnp.float32),
    )(xs2, hfull, hfull, wih3, whh3, bih3, bhh3)


def _bmm_body(inp_ref, w_ref, wg_ref, d0_ref, d1_ref, h_ref, g_ref):
    x = lax.dot_general(inp_ref[0], w_ref[0], (((1,), (0,)), ((), ())),
                        preferred_element_type=jnp.float32)
    h = lax.dot_general(x, wg_ref[...], (((1,), (0,)), ((), ())),
                        preferred_element_type=jnp.float32)
    deg = d0_ref[0] + d1_ref[0] + 2.0
    dis = lax.rsqrt(deg)
    h_ref[0] = h
    g_ref[0] = h * dis


_NB = 2000


def _bmm(inp, w3, wg, deg0, deg1):
    return pl.pallas_call(
        _bmm_body,
        grid=(B, N // _NB),
        in_specs=[
            pl.BlockSpec((1, _NB, C), lambda b, i: (b, i, 0)),
            pl.BlockSpec((1, C, H), lambda b, i: (b, 0, 0)),
            pl.BlockSpec((H, O), lambda b, i: (0, 0)),
            pl.BlockSpec((1, _NB, 1), lambda b, i: (b, i, 0)),
            pl.BlockSpec((1, _NB, 1), lambda b, i: (b, i, 0)),
        ],
        out_specs=[
            pl.BlockSpec((1, _NB, O), lambda b, i: (b, i, 0)),
            pl.BlockSpec((1, _NB, O), lambda b, i: (b, i, 0)),
        ],
        out_shape=[
            jax.ShapeDtypeStruct((B, N, O), jnp.float32),
            jax.ShapeDtypeStruct((B, N, O), jnp.float32),
        ],
    )(inp, w3, wg, deg0, deg1)


def _fin_body(a0_ref, a1_ref, d0_ref, d1_ref, h_ref, bg_ref, out_ref):
    deg = d0_ref[0] + d1_ref[0] + 2.0
    dis = lax.rsqrt(deg)
    acc = a0_ref[0] + a1_ref[0]
    out_ref[0] = dis * acc + (2.0 * dis * dis) * h_ref[0] + bg_ref[...]


def _fin(acc0, acc1, deg0, deg1, h, bg2):
    return pl.pallas_call(
        _fin_body,
        grid=(B, N // _NB),
        in_specs=[
            pl.BlockSpec((1, _NB, O), lambda b, i: (b, i, 0)),
            pl.BlockSpec((1, _NB, O), lambda b, i: (b, i, 0)),
            pl.BlockSpec((1, _NB, 1), lambda b, i: (b, i, 0)),
            pl.BlockSpec((1, _NB, 1), lambda b, i: (b, i, 0)),
            pl.BlockSpec((1, _NB, O), lambda b, i: (b, i, 0)),
            pl.BlockSpec((1, O), lambda b, i: (0, 0)),
        ],
        out_specs=pl.BlockSpec((1, _NB, O), lambda b, i: (b, i, 0)),
        out_shape=jax.ShapeDtypeStruct((B, N, O), jnp.float32),
    )(acc0, acc1, deg0, deg1, h, bg2)


_NW = 32
_EW = E // _NW
_CH = 80
_NCH = _EW // _CH
_WB = NN // 8
_SB = 1000

@functools.cache
def _sc_kernels():
    mesh = plsc.VectorSubcoreMesh(core_axis_name="c", subcore_axis_name="s")

    @functools.partial(
        pl.kernel,
        mesh=mesh,
        out_type=jax.ShapeDtypeStruct((2 * NN,), jnp.float32),
        scratch_types=[
            pltpu.VMEM((_CH,), jnp.int32),
            pltpu.VMEM((_CH,), jnp.float32),
            pltpu.VMEM((_WB,), jnp.float32),
            pltpu.VMEM_SHARED((NN,), jnp.float32),
        ],
    )
    def _sc_deg(dst_hbm, out_hbm, idx_v, ones_v, zbuf, deg_sh):
        cid = lax.axis_index("c")
        sid = lax.axis_index("s")
        wid = sid * 2 + cid
        for j in range(_CH // 16):
            ones_v[pl.ds(j * 16, 16)] = jnp.ones((16,), jnp.float32)

        def zb(j, carry):
            zbuf[pl.ds(pl.multiple_of(j * 16, 8), 16)] = (
                jnp.zeros((16,), jnp.float32))
            return carry

        lax.fori_loop(0, _WB // 16, zb, 0)

        @pl.when(sid < 8)
        def _():
            off = pl.multiple_of(sid * _WB, 8)
            pltpu.sync_copy(zbuf, deg_sh.at[pl.ds(off, _WB)])

        plsc.subcore_barrier()
        base = wid * _EW

        def body(i, carry):
            eoff = pl.multiple_of(base + i * _CH, 8)
            pltpu.sync_copy(dst_hbm.at[pl.ds(eoff, _CH)], idx_v)
            pltpu.sync_copy(ones_v, deg_sh.at[idx_v], add=True)
            return carry

        lax.fori_loop(0, _NCH, body, 0)
        plsc.subcore_barrier()

        @pl.when(sid < 8)
        def _():
            off = pl.multiple_of(sid * _WB, 8)
            ooff = pl.multiple_of(cid * NN + sid * _WB, 8)
            pltpu.sync_copy(deg_sh.at[pl.ds(off, _WB)], zbuf)
            pltpu.sync_copy(zbuf, out_hbm.at[pl.ds(ooff, _WB)])

    @functools.partial(
        pl.kernel,
        mesh=mesh,
        out_type=jax.ShapeDtypeStruct((2, NN, O), jnp.float32),
        scratch_types=[
            pltpu.VMEM((_CH,), jnp.int32),
            pltpu.VMEM((_CH,), jnp.int32),
            pltpu.VMEM((_CH, O), jnp.float32),
            pltpu.VMEM((_SB, O), jnp.float32),
            pltpu.VMEM_SHARED((NN, O), jnp.float32),
            pltpu.SemaphoreType.DMA,
        ],
        compiler_params=pltpu.CompilerParams(use_tc_tiling_on_sc=False),
    )
    def _sc_scatter(src_hbm, dst_hbm, g_hbm, out_hbm,
                    src_v, dst_v, rows_v, stage, acc_sh, sem):
        cid = lax.axis_index("c")
        sid = lax.axis_index("s")
        wid = sid * 2 + cid

        def zb(i, carry):
            stage[i, pl.ds(0, 16)] = jnp.zeros((16,), jnp.float32)
            stage[i, pl.ds(16, 16)] = jnp.zeros((16,), jnp.float32)
            return carry

        lax.fori_loop(0, _SB, zb, 0)

        @pl.when(sid < 8)
        def _():
            def zc(j, carry):
                off = pl.multiple_of(sid * _WB + j * _SB, 8)
                pltpu.sync_copy(stage, acc_sh.at[pl.ds(off, _SB)])
                return carry

            lax.fori_loop(0, _WB // _SB, zc, 0)

        plsc.subcore_barrier()
        base = wid * _EW

        def body(i, carry):
            eoff = pl.multiple_of(base + i * _CH, 8)
            pltpu.sync_copy(src_hbm.at[pl.ds(eoff, _CH)], src_v)
            pltpu.sync_copy(dst_hbm.at[pl.ds(eoff, _CH)], dst_v)
            pltpu.async_copy(g_hbm.at[src_v], rows_v, sem).wait()
            pltpu.sync_copy(rows_v, acc_sh.at[dst_v], add=True)
            return carry

        lax.fori_loop(0, _NCH, body, 0)
        plsc.subcore_barrier()

        @pl.when(sid < 8)
        def _():
            def wb(j, carry):
                off = pl.multiple_of(sid * _WB + j * _SB, 8)
                pltpu.sync_copy(acc_sh.at[pl.ds(off, _SB)], stage)
                pltpu.sync_copy(stage, out_hbm.at[cid, pl.ds(off, _SB)])
                return carry

            lax.fori_loop(0, _WB // _SB, wb, 0)

    return _sc_deg, _sc_scatter


def kernel(input, edge_index, last_graph_weights, p, Wih, Whh, bih, bhh, Wg,
           bg):
    src = edge_index[0]
    dst = edge_index[1]

    inp2 = input.reshape(NN, C)
    y2 = _proj(inp2, p.reshape(1, C), p.reshape(C, 1)).reshape(B, N)
    xs = _topk(input, y2)

    w = _gru(xs.reshape(B, GI), last_graph_weights,
             Wih.reshape(3, GH, GI), Whh.reshape(3, GH, GH),
             bih.reshape(3, 1, GH), bhh.reshape(3, 1, GH))

    sc_deg, sc_scatter = _sc_kernels()
    degp = sc_deg(dst).reshape(2, NN)
    deg0 = degp[0].reshape(B, N, 1)
    deg1 = degp[1].reshape(B, N, 1)

    h, g = _bmm(input, w.reshape(B, C, H), Wg, deg0, deg1)

    accp = sc_scatter(src, dst, g.reshape(NN, O))

    out = _fin(accp[0].reshape(B, N, O), accp[1].reshape(B, N, O),
               deg0, deg1, h, bg.reshape(1, O))
    return out, w.reshape(B, C, H)


# speedup vs baseline: 49.1928x; 2.5095x over previous
import functools

import jax
import jax.numpy as jnp
from jax import lax
from jax.experimental import pallas as pl
from jax.experimental.pallas import tpu as pltpu
from jax.experimental.pallas import tpu_sc as plsc

B, N, C, H, O, K = 4, 10000, 64, 32, 32, 16
E = 1280000
NN = B * N
GH = C * H
GI = K * C


def _proj_body(inp_ref, prow_ref, pcol_ref, y_ref):
    prow = prow_ref[...]
    pn = lax.rsqrt(jnp.sum(prow * prow))
    y = lax.dot_general(inp_ref[...], pcol_ref[...],
                        (((1,), (0,)), ((), ())),
                        preferred_element_type=jnp.float32)
    y_ref[...] = y * pn


def _proj(inp2, prow, pcol):
    bn = 8000
    return pl.pallas_call(
        _proj_body,
        grid=(NN // bn,),
        in_specs=[
            pl.BlockSpec((bn, C), lambda i: (i, 0)),
            pl.BlockSpec((1, C), lambda i: (0, 0)),
            pl.BlockSpec((C, 1), lambda i: (0, 0)),
        ],
        out_specs=pl.BlockSpec((bn, 1), lambda i: (i, 0)),
        out_shape=jax.ShapeDtypeStruct((NN, 1), jnp.float32),
    )(inp2, prow, pcol)


def _topk_body(inp_ref, y_ref, xs_ref):
    y = y_ref[...]
    iota = lax.broadcasted_iota(jnp.int32, (B, N), 1)
    for k in range(K):
        m = jnp.max(y, axis=1, keepdims=True)
        idxv = jnp.min(jnp.where(y == m, iota, N), axis=1, keepdims=True)
        oh = (iota == idxv).astype(jnp.float32)
        for b in range(B):
            row = lax.dot_general(oh[b:b + 1], inp_ref[b],
                                  (((1,), (0,)), ((), ())),
                                  preferred_element_type=jnp.float32)
            xs_ref[b, pl.ds(k, 1), :] = row * jnp.tanh(m[b:b + 1, 0:1])
        y = jnp.where(iota == idxv, -jnp.inf, y)


def _topk(inp, y2):
    return pl.pallas_call(
        _topk_body,
        out_shape=jax.ShapeDtypeStruct((B, K, C), jnp.float32),
    )(inp, y2)


_BJ = 256


def _gru_body(xs_ref, hf_ref, hb_ref, wih_ref, whh_ref, bih_ref, bhh_ref,
              w_ref):
    xs = xs_ref[...]
    hf = hf_ref[...]
    g = []
    for gi in range(3):
        a = lax.dot_general(xs, wih_ref[gi], (((1,), (1,)), ((), ())),
                            preferred_element_type=jnp.float32)
        a = a + bih_ref[gi]
        c = lax.dot_general(hf, whh_ref[gi], (((1,), (1,)), ((), ())),
                            preferred_element_type=jnp.float32)
        c = c + bhh_ref[gi]
        g.append((a, c))
    r = jax.nn.sigmoid(g[0][0] + g[0][1])
    z = jax.nn.sigmoid(g[1][0] + g[1][1])
    n = jnp.tanh(g[2][0] + r * g[2][1])
    w_ref[...] = (1.0 - z) * n + z * hb_ref[...]


def _gru(xs2, hfull, wih3, whh3, bih3, bhh3):
    return pl.pallas_call(
        _gru_body,
        grid=(GH // _BJ,),
        in_specs=[
            pl.BlockSpec((B, GI), lambda j: (0, 0)),
            pl.BlockSpec((B, GH), lambda j: (0, 0)),
            pl.BlockSpec((B, _BJ), lambda j: (0, j)),
            pl.BlockSpec((3, _BJ, GI), lambda j: (0, j, 0)),
            pl.BlockSpec((3, _BJ, GH), lambda j: (0, j, 0)),
            pl.BlockSpec((3, 1, _BJ), lambda j: (0, 0, j)),
            pl.BlockSpec((3, 1, _BJ), lambda j: (0, 0, j)),
        ],
        out_specs=pl.BlockSpec((B, _BJ), lambda j: (0, j)),
        out_shape=jax.ShapeDtypeStruct((B, GH), jnp.float32),
    )(xs2, hfull, hfull, wih3, whh3, bih3, bhh3)


def _bmm_body(inp_ref, w_ref, wg_ref, d0_ref, d1_ref, h_ref, g_ref):
    x = lax.dot_general(inp_ref[0], w_ref[0], (((1,), (0,)), ((), ())),
                        preferred_element_type=jnp.float32)
    h = lax.dot_general(x, wg_ref[...], (((1,), (0,)), ((), ())),
                        preferred_element_type=jnp.float32)
    deg = d0_ref[0] + d1_ref[0] + 2.0
    dis = lax.rsqrt(deg)
    h_ref[0] = h
    g_ref[0] = h * dis


_NB = 2000


def _bmm(inp, w3, wg, deg0, deg1):
    return pl.pallas_call(
        _bmm_body,
        grid=(B, N // _NB),
        in_specs=[
            pl.BlockSpec((1, _NB, C), lambda b, i: (b, i, 0)),
            pl.BlockSpec((1, C, H), lambda b, i: (b, 0, 0)),
            pl.BlockSpec((H, O), lambda b, i: (0, 0)),
            pl.BlockSpec((1, _NB, 1), lambda b, i: (b, i, 0)),
            pl.BlockSpec((1, _NB, 1), lambda b, i: (b, i, 0)),
        ],
        out_specs=[
            pl.BlockSpec((1, _NB, O), lambda b, i: (b, i, 0)),
            pl.BlockSpec((1, _NB, O), lambda b, i: (b, i, 0)),
        ],
        out_shape=[
            jax.ShapeDtypeStruct((B, N, O), jnp.float32),
            jax.ShapeDtypeStruct((B, N, O), jnp.float32),
        ],
    )(inp, w3, wg, deg0, deg1)


def _fin_body(a0_ref, a1_ref, d0_ref, d1_ref, h_ref, bg_ref, out_ref):
    deg = d0_ref[0] + d1_ref[0] + 2.0
    dis = lax.rsqrt(deg)
    acc = a0_ref[0] + a1_ref[0]
    out_ref[0] = dis * acc + (2.0 * dis * dis) * h_ref[0] + bg_ref[...]


def _fin(acc0, acc1, deg0, deg1, h, bg2):
    return pl.pallas_call(
        _fin_body,
        grid=(B, N // _NB),
        in_specs=[
            pl.BlockSpec((1, _NB, O), lambda b, i: (b, i, 0)),
            pl.BlockSpec((1, _NB, O), lambda b, i: (b, i, 0)),
            pl.BlockSpec((1, _NB, 1), lambda b, i: (b, i, 0)),
            pl.BlockSpec((1, _NB, 1), lambda b, i: (b, i, 0)),
            pl.BlockSpec((1, _NB, O), lambda b, i: (b, i, 0)),
            pl.BlockSpec((1, O), lambda b, i: (0, 0)),
        ],
        out_specs=pl.BlockSpec((1, _NB, O), lambda b, i: (b, i, 0)),
        out_shape=jax.ShapeDtypeStruct((B, N, O), jnp.float32),
    )(acc0, acc1, deg0, deg1, h, bg2)


_NW = 32
_EW = E // _NW
_CH = 80
_ER = E // _CH
_RW = _EW // _CH
_G = 5
_NG = _RW // _G
_GD = 10
_WB = NN // 8
_SB = 500

@functools.cache
def _sc_kernels():
    mesh = plsc.VectorSubcoreMesh(core_axis_name="c", subcore_axis_name="s")

    @functools.partial(
        pl.kernel,
        mesh=mesh,
        out_type=jax.ShapeDtypeStruct((2 * NN,), jnp.float32),
        scratch_types=[
            pltpu.VMEM((_GD, _CH), jnp.int32),
            pltpu.VMEM((_GD, _CH), jnp.int32),
            pltpu.VMEM((_CH,), jnp.float32),
            pltpu.VMEM((_WB,), jnp.float32),
            pltpu.VMEM_SHARED((NN,), jnp.float32),
            pltpu.SemaphoreType.DMA,
            pltpu.SemaphoreType.DMA,
        ],
        compiler_params=pltpu.CompilerParams(use_tc_tiling_on_sc=False),
    )
    def _sc_deg(dst_hbm, out_hbm, idx_a, idx_b, ones_v, zbuf, deg_sh,
                sem_a, sem_b):
        cid = lax.axis_index("c")
        sid = lax.axis_index("s")
        wid = sid * 2 + cid
        for j in range(_CH // 16):
            ones_v[pl.ds(j * 16, 16)] = jnp.ones((16,), jnp.float32)

        def zb(j, carry):
            zbuf[pl.ds(pl.multiple_of(j * 16, 8), 16)] = (
                jnp.zeros((16,), jnp.float32))
            return carry

        lax.fori_loop(0, _WB // 16, zb, 0)

        @pl.when(sid < 8)
        def _():
            off = pl.multiple_of(sid * _WB, 8)
            pltpu.sync_copy(zbuf, deg_sh.at[pl.ds(off, _WB)])

        plsc.subcore_barrier()
        rbase = wid * _RW
        ngd = _RW // _GD

        def dfire(gg, idx_v, sem):
            pltpu.sync_copy(dst_hbm.at[pl.ds(rbase + gg * _GD, _GD)], idx_v)
            for j in range(_GD):
                pltpu.async_copy(ones_v, deg_sh.at[idx_v.at[j]], sem,
                                 add=True)

        def ddrain(idx_v, sem):
            for j in range(_GD):
                pltpu.make_async_copy(ones_v, deg_sh.at[idx_v.at[j]],
                                      sem).wait()

        dfire(0, idx_a, sem_a)

        def body(it, carry):
            gg = it * 2
            dfire(gg + 1, idx_b, sem_b)
            ddrain(idx_a, sem_a)

            @pl.when(it < ngd // 2 - 1)
            def _():
                dfire(gg + 2, idx_a, sem_a)

            ddrain(idx_b, sem_b)
            return carry

        lax.fori_loop(0, ngd // 2, body, 0)
        plsc.subcore_barrier()

        @pl.when(sid < 8)
        def _():
            off = pl.multiple_of(sid * _WB, 8)
            ooff = pl.multiple_of(cid * NN + sid * _WB, 8)
            pltpu.sync_copy(deg_sh.at[pl.ds(off, _WB)], zbuf)
            pltpu.sync_copy(zbuf, out_hbm.at[pl.ds(ooff, _WB)])

    @functools.partial(
        pl.kernel,
        mesh=mesh,
        out_type=jax.ShapeDtypeStruct((2, NN, O), jnp.float32),
        scratch_types=[
            pltpu.VMEM((_G, _CH), jnp.int32),
            pltpu.VMEM((_G, _CH), jnp.int32),
            pltpu.VMEM((_G, _CH), jnp.int32),
            pltpu.VMEM((_G, _CH), jnp.int32),
            pltpu.VMEM((_G, _CH, O), jnp.float32),
            pltpu.VMEM((_G, _CH, O), jnp.float32),
            pltpu.VMEM((_SB, O), jnp.float32),
            pltpu.VMEM_SHARED((NN, O), jnp.float32),
            pltpu.SemaphoreType.DMA,
            pltpu.SemaphoreType.DMA,
        ],
        compiler_params=pltpu.CompilerParams(use_tc_tiling_on_sc=False),
    )
    def _sc_scatter(src_hbm, dst_hbm, g_hbm, out_hbm,
                    src_a, dst_a, src_b, dst_b, rows_a, rows_b,
                    stage, acc_sh, sem_a, sem_b):
        cid = lax.axis_index("c")
        sid = lax.axis_index("s")
        wid = sid * 2 + cid

        def zb(i, carry):
            stage[i, pl.ds(0, 16)] = jnp.zeros((16,), jnp.float32)
            stage[i, pl.ds(16, 16)] = jnp.zeros((16,), jnp.float32)
            return carry

        lax.fori_loop(0, _SB, zb, 0)

        @pl.when(sid < 8)
        def _():
            def zc(j, carry):
                off = pl.multiple_of(sid * _WB + j * _SB, 8)
                pltpu.sync_copy(stage, acc_sh.at[pl.ds(off, _SB)])
                return carry

            lax.fori_loop(0, _WB // _SB, zc, 0)

        plsc.subcore_barrier()
        rbase = wid * _RW

        def fire(gg, src_v, dst_v, rows_v, sem):
            roff = rbase + gg * _G
            pltpu.sync_copy(src_hbm.at[pl.ds(roff, _G)], src_v)
            pltpu.sync_copy(dst_hbm.at[pl.ds(roff, _G)], dst_v)
            for j in range(_G):
                pltpu.async_copy(g_hbm.at[src_v.at[j]], rows_v.at[j], sem)

        def finish(src_v, dst_v, rows_v, sem):
            for j in range(_G):
                pltpu.make_async_copy(g_hbm.at[src_v.at[j]], rows_v.at[j],
                                      sem).wait()
            for j in range(_G):
                pltpu.sync_copy(rows_v.at[j], acc_sh.at[dst_v.at[j]],
                                add=True)

        fire(0, src_a, dst_a, rows_a, sem_a)

        def body(it, carry):
            gg = it * 2
            fire(gg + 1, src_b, dst_b, rows_b, sem_b)
            finish(src_a, dst_a, rows_a, sem_a)

            @pl.when(it < _NG // 2 - 1)
            def _():
                fire(gg + 2, src_a, dst_a, rows_a, sem_a)

            finish(src_b, dst_b, rows_b, sem_b)
            return carry

        lax.fori_loop(0, _NG // 2, body, 0)
        plsc.subcore_barrier()

        @pl.when(sid < 8)
        def _():
            def wb(j, carry):
                off = pl.multiple_of(sid * _WB + j * _SB, 8)
                pltpu.sync_copy(acc_sh.at[pl.ds(off, _SB)], stage)
                pltpu.sync_copy(stage, out_hbm.at[cid, pl.ds(off, _SB)])
                return carry

            lax.fori_loop(0, _WB // _SB, wb, 0)

    return _sc_deg, _sc_scatter


def kernel(input, edge_index, last_graph_weights, p, Wih, Whh, bih, bhh, Wg,
           bg):
    src = edge_index[0].reshape(_ER, _CH)
    dst = edge_index[1].reshape(_ER, _CH)

    inp2 = input.reshape(NN, C)
    y2 = _proj(inp2, p.reshape(1, C), p.reshape(C, 1)).reshape(B, N)
    xs = _topk(input, y2)

    w = _gru(xs.reshape(B, GI), last_graph_weights,
             Wih.reshape(3, GH, GI), Whh.reshape(3, GH, GH),
             bih.reshape(3, 1, GH), bhh.reshape(3, 1, GH))

    sc_deg, sc_scatter = _sc_kernels()
    degp = sc_deg(dst).reshape(2, NN)
    deg0 = degp[0].reshape(B, N, 1)
    deg1 = degp[1].reshape(B, N, 1)

    h, g = _bmm(input, w.reshape(B, C, H), Wg, deg0, deg1)

    accp = sc_scatter(src, dst, g.reshape(NN, O))

    out = _fin(accp[0].reshape(B, N, O), accp[1].reshape(B, N, O),
               deg0, deg1, h, bg.reshape(1, O))
    return out, w.reshape(B, C, H)


# speedup vs baseline: 52.6083x; 1.0694x over previous
import functools

import jax
import jax.numpy as jnp
from jax import lax
from jax.experimental import pallas as pl
from jax.experimental.pallas import tpu as pltpu
from jax.experimental.pallas import tpu_sc as plsc

B, N, C, H, O, K = 4, 10000, 64, 32, 32, 16
E = 1280000
NN = B * N
GH = C * H
GI = K * C


def _proj_body(inp_ref, prow_ref, pcol_ref, y_ref):
    prow = prow_ref[...]
    pn = lax.rsqrt(jnp.sum(prow * prow))
    y = lax.dot_general(inp_ref[...], pcol_ref[...],
                        (((1,), (0,)), ((), ())),
                        preferred_element_type=jnp.float32)
    y_ref[...] = y * pn


def _proj(inp2, prow, pcol):
    bn = 8000
    return pl.pallas_call(
        _proj_body,
        grid=(NN // bn,),
        in_specs=[
            pl.BlockSpec((bn, C), lambda i: (i, 0)),
            pl.BlockSpec((1, C), lambda i: (0, 0)),
            pl.BlockSpec((C, 1), lambda i: (0, 0)),
        ],
        out_specs=pl.BlockSpec((bn, 1), lambda i: (i, 0)),
        out_shape=jax.ShapeDtypeStruct((NN, 1), jnp.float32),
    )(inp2, prow, pcol)


def _topk_body(inp_ref, y_ref, xs_ref):
    y = y_ref[...]
    iota = lax.broadcasted_iota(jnp.int32, (B, N), 1)
    for k in range(K):
        m = jnp.max(y, axis=1, keepdims=True)
        idxv = jnp.min(jnp.where(y == m, iota, N), axis=1, keepdims=True)
        oh = (iota == idxv).astype(jnp.float32)
        for b in range(B):
            row = lax.dot_general(oh[b:b + 1], inp_ref[b],
                                  (((1,), (0,)), ((), ())),
                                  preferred_element_type=jnp.float32)
            xs_ref[b, pl.ds(k, 1), :] = row * jnp.tanh(m[b:b + 1, 0:1])
        y = jnp.where(iota == idxv, -jnp.inf, y)


def _topk(inp, y2):
    return pl.pallas_call(
        _topk_body,
        out_shape=jax.ShapeDtypeStruct((B, K, C), jnp.float32),
    )(inp, y2)


_BJ = 256


def _gru_body(xs_ref, hf_ref, hb_ref, wih_ref, whh_ref, bih_ref, bhh_ref,
              w_ref):
    xs = xs_ref[...]
    hf = hf_ref[...]
    g = []
    for gi in range(3):
        a = lax.dot_general(xs, wih_ref[gi], (((1,), (1,)), ((), ())),
                            preferred_element_type=jnp.float32)
        a = a + bih_ref[gi]
        c = lax.dot_general(hf, whh_ref[gi], (((1,), (1,)), ((), ())),
                            preferred_element_type=jnp.float32)
        c = c + bhh_ref[gi]
        g.append((a, c))
    r = jax.nn.sigmoid(g[0][0] + g[0][1])
    z = jax.nn.sigmoid(g[1][0] + g[1][1])
    n = jnp.tanh(g[2][0] + r * g[2][1])
    w_ref[...] = (1.0 - z) * n + z * hb_ref[...]


def _gru(xs2, hfull, wih3, whh3, bih3, bhh3):
    return pl.pallas_call(
        _gru_body,
        grid=(GH // _BJ,),
        in_specs=[
            pl.BlockSpec((B, GI), lambda j: (0, 0)),
            pl.BlockSpec((B, GH), lambda j: (0, 0)),
            pl.BlockSpec((B, _BJ), lambda j: (0, j)),
            pl.BlockSpec((3, _BJ, GI), lambda j: (0, j, 0)),
            pl.BlockSpec((3, _BJ, GH), lambda j: (0, j, 0)),
            pl.BlockSpec((3, 1, _BJ), lambda j: (0, 0, j)),
            pl.BlockSpec((3, 1, _BJ), lambda j: (0, 0, j)),
        ],
        out_specs=pl.BlockSpec((B, _BJ), lambda j: (0, j)),
        out_shape=jax.ShapeDtypeStruct((B, GH), jnp.float32),
    )(xs2, hfull, hfull, wih3, whh3, bih3, bhh3)


def _bmm_body(inp_ref, w_ref, wg_ref, d0_ref, d1_ref, h_ref, g_ref):
    x = lax.dot_general(inp_ref[0], w_ref[0], (((1,), (0,)), ((), ())),
                        preferred_element_type=jnp.float32)
    h = lax.dot_general(x, wg_ref[...], (((1,), (0,)), ((), ())),
                        preferred_element_type=jnp.float32)
    deg = d0_ref[0] + d1_ref[0] + 2.0
    dis = lax.rsqrt(deg)
    h_ref[0] = h
    g_ref[0] = h * dis


_NB = 2000


def _bmm(inp, w3, wg, deg0, deg1):
    return pl.pallas_call(
        _bmm_body,
        grid=(B, N // _NB),
        in_specs=[
            pl.BlockSpec((1, _NB, C), lambda b, i: (b, i, 0)),
            pl.BlockSpec((1, C, H), lambda b, i: (b, 0, 0)),
            pl.BlockSpec((H, O), lambda b, i: (0, 0)),
            pl.BlockSpec((1, _NB, 1), lambda b, i: (b, i, 0)),
            pl.BlockSpec((1, _NB, 1), lambda b, i: (b, i, 0)),
        ],
        out_specs=[
            pl.BlockSpec((1, _NB, O), lambda b, i: (b, i, 0)),
            pl.BlockSpec((1, _NB, O), lambda b, i: (b, i, 0)),
        ],
        out_shape=[
            jax.ShapeDtypeStruct((B, N, O), jnp.float32),
            jax.ShapeDtypeStruct((B, N, O), jnp.float32),
        ],
    )(inp, w3, wg, deg0, deg1)


def _fin_body(a0_ref, a1_ref, d0_ref, d1_ref, h_ref, bg_ref, out_ref):
    deg = d0_ref[0] + d1_ref[0] + 2.0
    dis = lax.rsqrt(deg)
    acc = a0_ref[0] + a1_ref[0]
    out_ref[0] = dis * acc + (2.0 * dis * dis) * h_ref[0] + bg_ref[...]


def _fin(acc0, acc1, deg0, deg1, h, bg2):
    return pl.pallas_call(
        _fin_body,
        grid=(B, N // _NB),
        in_specs=[
            pl.BlockSpec((1, _NB, O), lambda b, i: (b, i, 0)),
            pl.BlockSpec((1, _NB, O), lambda b, i: (b, i, 0)),
            pl.BlockSpec((1, _NB, 1), lambda b, i: (b, i, 0)),
            pl.BlockSpec((1, _NB, 1), lambda b, i: (b, i, 0)),
            pl.BlockSpec((1, _NB, O), lambda b, i: (b, i, 0)),
            pl.BlockSpec((1, O), lambda b, i: (0, 0)),
        ],
        out_specs=pl.BlockSpec((1, _NB, O), lambda b, i: (b, i, 0)),
        out_shape=jax.ShapeDtypeStruct((B, N, O), jnp.float32),
    )(acc0, acc1, deg0, deg1, h, bg2)


_NW = 32
_EW = E // _NW
_CH = 80
_ER = E // _CH
_RW = _EW // _CH
_GD = 10
_C2 = 125
_E2 = E // _C2
_R2 = _E2 // _NW
_G = 5
_NG = _R2 // _G
_WB = NN // 8
_WT = NN // 16
_SB = 250

@functools.cache
def _sc_kernels():
    mesh = plsc.VectorSubcoreMesh(core_axis_name="c", subcore_axis_name="s")

    @functools.partial(
        pl.kernel,
        mesh=mesh,
        out_type=jax.ShapeDtypeStruct((2 * NN,), jnp.float32),
        scratch_types=[
            pltpu.VMEM((_GD, _CH), jnp.int32),
            pltpu.VMEM((_GD, _CH), jnp.int32),
            pltpu.VMEM((_CH,), jnp.float32),
            pltpu.VMEM((_WB,), jnp.float32),
            pltpu.VMEM_SHARED((NN,), jnp.float32),
            pltpu.SemaphoreType.DMA,
            pltpu.SemaphoreType.DMA,
        ],
        compiler_params=pltpu.CompilerParams(use_tc_tiling_on_sc=False),
    )
    def _sc_deg(dst_hbm, out_hbm, idx_a, idx_b, ones_v, zbuf, deg_sh,
                sem_a, sem_b):
        cid = lax.axis_index("c")
        sid = lax.axis_index("s")
        wid = sid * 2 + cid
        for j in range(_CH // 16):
            ones_v[pl.ds(j * 16, 16)] = jnp.ones((16,), jnp.float32)

        def zb(j, carry):
            zbuf[pl.ds(pl.multiple_of(j * 16, 8), 16)] = (
                jnp.zeros((16,), jnp.float32))
            return carry

        lax.fori_loop(0, _WB // 16, zb, 0)

        @pl.when(sid < 8)
        def _():
            off = pl.multiple_of(sid * _WB, 8)
            pltpu.sync_copy(zbuf, deg_sh.at[pl.ds(off, _WB)])

        plsc.subcore_barrier()
        rbase = wid * _RW
        ngd = _RW // _GD

        def dfire(gg, idx_v, sem):
            pltpu.sync_copy(dst_hbm.at[pl.ds(rbase + gg * _GD, _GD)], idx_v)
            for j in range(_GD):
                pltpu.async_copy(ones_v, deg_sh.at[idx_v.at[j]], sem,
                                 add=True)

        def ddrain(idx_v, sem):
            for j in range(_GD):
                pltpu.make_async_copy(ones_v, deg_sh.at[idx_v.at[j]],
                                      sem).wait()

        dfire(0, idx_a, sem_a)

        def body(it, carry):
            gg = it * 2
            dfire(gg + 1, idx_b, sem_b)
            ddrain(idx_a, sem_a)

            @pl.when(it < ngd // 2 - 1)
            def _():
                dfire(gg + 2, idx_a, sem_a)

            ddrain(idx_b, sem_b)
            return carry

        lax.fori_loop(0, ngd // 2, body, 0)
        plsc.subcore_barrier()

        @pl.when(sid < 8)
        def _():
            off = pl.multiple_of(sid * _WB, 8)
            ooff = pl.multiple_of(cid * NN + sid * _WB, 8)
            pltpu.sync_copy(deg_sh.at[pl.ds(off, _WB)], zbuf)
            pltpu.sync_copy(zbuf, out_hbm.at[pl.ds(ooff, _WB)])

    @functools.partial(
        pl.kernel,
        mesh=mesh,
        out_type=jax.ShapeDtypeStruct((2, NN, O), jnp.float32),
        scratch_types=[
            pltpu.VMEM((2, _G, _C2), jnp.int32),
            pltpu.VMEM((2, _G, _C2), jnp.int32),
            pltpu.VMEM((_G, _C2, O), jnp.float32),
            pltpu.VMEM((_G, _C2, O), jnp.float32),
            pltpu.VMEM((_SB, O), jnp.float32),
            pltpu.VMEM_SHARED((NN, O), jnp.float32),
            pltpu.SemaphoreType.DMA,
            pltpu.SemaphoreType.DMA,
            pltpu.SemaphoreType.DMA,
            pltpu.SemaphoreType.DMA,
        ],
        compiler_params=pltpu.CompilerParams(use_tc_tiling_on_sc=False),
    )
    def _sc_scatter(ei_hbm, g_hbm, out_hbm,
                    idx_a, idx_b, rows_a, rows_b,
                    stage, acc_sh, gsem_a, gsem_b, ssem_a, ssem_b):
        cid = lax.axis_index("c")
        sid = lax.axis_index("s")
        wid = sid * 2 + cid

        def zb(i, carry):
            stage[i, pl.ds(0, 16)] = jnp.zeros((16,), jnp.float32)
            stage[i, pl.ds(16, 16)] = jnp.zeros((16,), jnp.float32)
            return carry

        lax.fori_loop(0, _SB, zb, 0)

        def zc(j, carry):
            pltpu.sync_copy(stage, acc_sh.at[pl.ds(sid * _WT + j * _SB,
                                                   _SB)])
            return carry

        lax.fori_loop(0, _WT // _SB, zc, 0)
        plsc.subcore_barrier()
        rbase = wid * _R2

        def drain_scat(idx_v, rows_v, ssem):
            for j in range(_G):
                pltpu.make_async_copy(rows_v.at[j],
                                      acc_sh.at[idx_v.at[1].at[j]],
                                      ssem).wait()

        def fire(gg, idx_v, rows_v, gsem, ssem):
            @pl.when(gg >= 2)
            def _():
                drain_scat(idx_v, rows_v, ssem)

            roff = rbase + gg * _G
            pltpu.sync_copy(ei_hbm.at[:, pl.ds(roff, _G)], idx_v)
            for j in range(_G):
                pltpu.async_copy(g_hbm.at[idx_v.at[0].at[j]],
                                 rows_v.at[j], gsem)

        def finish(idx_v, rows_v, gsem, ssem):
            for j in range(_G):
                pltpu.make_async_copy(g_hbm.at[idx_v.at[0].at[j]],
                                      rows_v.at[j], gsem).wait()
            for j in range(_G):
                pltpu.async_copy(rows_v.at[j], acc_sh.at[idx_v.at[1].at[j]],
                                 ssem, add=True)

        fire(0, idx_a, rows_a, gsem_a, ssem_a)

        def body(it, carry):
            gg = it * 2
            fire(gg + 1, idx_b, rows_b, gsem_b, ssem_b)
            finish(idx_a, rows_a, gsem_a, ssem_a)

            @pl.when(it < _NG // 2 - 1)
            def _():
                fire(gg + 2, idx_a, rows_a, gsem_a, ssem_a)

            finish(idx_b, rows_b, gsem_b, ssem_b)
            return carry

        lax.fori_loop(0, _NG // 2, body, 0)
        drain_scat(idx_a, rows_a, ssem_a)
        drain_scat(idx_b, rows_b, ssem_b)
        plsc.subcore_barrier()

        def wb(j, carry):
            off = sid * _WT + j * _SB
            pltpu.sync_copy(acc_sh.at[pl.ds(off, _SB)], stage)
            pltpu.sync_copy(stage, out_hbm.at[cid, pl.ds(off, _SB)])
            return carry

        lax.fori_loop(0, _WT // _SB, wb, 0)

    return _sc_deg, _sc_scatter


def kernel(input, edge_index, last_graph_weights, p, Wih, Whh, bih, bhh, Wg,
           bg):
    dst = edge_index[1].reshape(_ER, _CH)
    ei3 = edge_index.reshape(2, _E2, _C2)

    inp2 = input.reshape(NN, C)
    y2 = _proj(inp2, p.reshape(1, C), p.reshape(C, 1)).reshape(B, N)
    xs = _topk(input, y2)

    w = _gru(xs.reshape(B, GI), last_graph_weights,
             Wih.reshape(3, GH, GI), Whh.reshape(3, GH, GH),
             bih.reshape(3, 1, GH), bhh.reshape(3, 1, GH))

    sc_deg, sc_scatter = _sc_kernels()
    degp = sc_deg(dst).reshape(2, NN)
    deg0 = degp[0].reshape(B, N, 1)
    deg1 = degp[1].reshape(B, N, 1)

    h, g = _bmm(input, w.reshape(B, C, H), Wg, deg0, deg1)

    accp = sc_scatter(ei3, g.reshape(NN, O))

    out = _fin(accp[0].reshape(B, N, O), accp[1].reshape(B, N, O),
               deg0, deg1, h, bg.reshape(1, O))
    return out, w.reshape(B, C, H)


# speedup vs baseline: 62.1820x; 1.1820x over previous
import functools

import jax
import jax.numpy as jnp
from jax import lax
from jax.experimental import pallas as pl
from jax.experimental.pallas import tpu as pltpu
from jax.experimental.pallas import tpu_sc as plsc

B, N, C, H, O, K = 4, 10000, 64, 32, 32, 16
E = 1280000
NN = B * N
GH = C * H
GI = K * C


def _proj_body(inp_ref, prow_ref, pcol_ref, y_ref):
    prow = prow_ref[...]
    pn = lax.rsqrt(jnp.sum(prow * prow))
    y = lax.dot_general(inp_ref[0], pcol_ref[...],
                        (((1,), (0,)), ((), ())),
                        preferred_element_type=jnp.float32)
    y_ref[...] = y * pn


_PB = 2000


def _proj(inp, prow, pcol):
    return pl.pallas_call(
        _proj_body,
        grid=(B, N // _PB),
        in_specs=[
            pl.BlockSpec((1, _PB, C), lambda b, i: (b, i, 0)),
            pl.BlockSpec((1, C), lambda b, i: (0, 0)),
            pl.BlockSpec((C, 1), lambda b, i: (0, 0)),
        ],
        out_specs=pl.BlockSpec((_PB, 1),
                               lambda b, i: (b * (N // _PB) + i, 0)),
        out_shape=jax.ShapeDtypeStruct((NN, 1), jnp.float32),
    )(inp, prow, pcol)


def _topk_body(inp_ref, y_ref, xs_ref):
    y = y_ref[...]
    iota = lax.broadcasted_iota(jnp.int32, (B, N), 1)
    for k in range(K):
        m = jnp.max(y, axis=1, keepdims=True)
        idxv = jnp.min(jnp.where(y == m, iota, N), axis=1, keepdims=True)
        oh = (iota == idxv).astype(jnp.float32)
        for b in range(B):
            row = lax.dot_general(oh[b:b + 1], inp_ref[b],
                                  (((1,), (0,)), ((), ())),
                                  preferred_element_type=jnp.float32)
            xs_ref[b, pl.ds(k, 1), :] = row * jnp.tanh(m[b:b + 1, 0:1])
        y = jnp.where(iota == idxv, -jnp.inf, y)


def _topk(inp, y2):
    return pl.pallas_call(
        _topk_body,
        out_shape=jax.ShapeDtypeStruct((B, K, C), jnp.float32),
    )(inp, y2)


_BJ = 256


def _gru_body(xs_ref, hf_ref, hb_ref, wih_ref, whh_ref, bih_ref, bhh_ref,
              w_ref):
    xs = xs_ref[...]
    hf = hf_ref[...]
    g = []
    for gi in range(3):
        a = lax.dot_general(xs, wih_ref[gi], (((1,), (1,)), ((), ())),
                            preferred_element_type=jnp.float32)
        a = a + bih_ref[gi]
        c = lax.dot_general(hf, whh_ref[gi], (((1,), (1,)), ((), ())),
                            preferred_element_type=jnp.float32)
        c = c + bhh_ref[gi]
        g.append((a, c))
    r = jax.nn.sigmoid(g[0][0] + g[0][1])
    z = jax.nn.sigmoid(g[1][0] + g[1][1])
    n = jnp.tanh(g[2][0] + r * g[2][1])
    w_ref[...] = (1.0 - z) * n + z * hb_ref[...]


def _gru(xs2, hfull, wih3, whh3, bih3, bhh3):
    return pl.pallas_call(
        _gru_body,
        grid=(GH // _BJ,),
        in_specs=[
            pl.BlockSpec((B, GI), lambda j: (0, 0)),
            pl.BlockSpec((B, GH), lambda j: (0, 0)),
            pl.BlockSpec((B, _BJ), lambda j: (0, j)),
            pl.BlockSpec((3, _BJ, GI), lambda j: (0, j, 0)),
            pl.BlockSpec((3, _BJ, GH), lambda j: (0, j, 0)),
            pl.BlockSpec((3, 1, _BJ), lambda j: (0, 0, j)),
            pl.BlockSpec((3, 1, _BJ), lambda j: (0, 0, j)),
        ],
        out_specs=pl.BlockSpec((B, _BJ), lambda j: (0, j)),
        out_shape=jax.ShapeDtypeStruct((B, GH), jnp.float32),
    )(xs2, hfull, hfull, wih3, whh3, bih3, bhh3)


_NB = 2000


def _bmm_body(inp_ref, w_ref, wg_ref, dis_ref, h_ref, g_ref):
    x = lax.dot_general(inp_ref[0], w_ref[0], (((1,), (0,)), ((), ())),
                        preferred_element_type=jnp.float32)
    h = lax.dot_general(x, wg_ref[...], (((1,), (0,)), ((), ())),
                        preferred_element_type=jnp.float32)
    h_ref[...] = h
    g_ref[...] = h * dis_ref[...]


def _bmm(inp, w3, wg, dis32):
    return pl.pallas_call(
        _bmm_body,
        grid=(B, N // _NB),
        in_specs=[
            pl.BlockSpec((1, _NB, C), lambda b, i: (b, i, 0)),
            pl.BlockSpec((1, C, H), lambda b, i: (b, 0, 0)),
            pl.BlockSpec((H, O), lambda b, i: (0, 0)),
            pl.BlockSpec((_NB, O), lambda b, i: (b * (N // _NB) + i, 0)),
        ],
        out_specs=[
            pl.BlockSpec((_NB, O), lambda b, i: (b * (N // _NB) + i, 0)),
            pl.BlockSpec((_NB, O), lambda b, i: (b * (N // _NB) + i, 0)),
        ],
        out_shape=[
            jax.ShapeDtypeStruct((NN, O), jnp.float32),
            jax.ShapeDtypeStruct((NN, O), jnp.float32),
        ],
    )(inp, w3, wg, dis32)


def _fin_body(acc_ref, dis_ref, h_ref, bg_ref, out_ref):
    dis = dis_ref[...]
    acc = acc_ref[0] + acc_ref[1]
    out_ref[...] = dis * acc + (2.0 * dis * dis) * h_ref[...] + bg_ref[...]


def _fin(accp, dis32, h, bg2):
    return pl.pallas_call(
        _fin_body,
        grid=(NN // _NB,),
        in_specs=[
            pl.BlockSpec((2, _NB, O), lambda i: (0, i, 0)),
            pl.BlockSpec((_NB, O), lambda i: (i, 0)),
            pl.BlockSpec((_NB, O), lambda i: (i, 0)),
            pl.BlockSpec((1, O), lambda i: (0, 0)),
        ],
        out_specs=pl.BlockSpec((_NB, O), lambda i: (i, 0)),
        out_shape=jax.ShapeDtypeStruct((NN, O), jnp.float32),
    )(accp, dis32, h, bg2)


_NW = 32
_EW = E // _NW
_CH = 80
_ER = E // _CH
_RW = _EW // _CH
_GD = 10
_C2 = 125
_E2 = E // _C2
_R2 = _E2 // _NW
_G = 5
_NG = _R2 // _G
_WB = NN // 8
_DT = 2000
_WT = NN // 16
_SB = 250

@functools.cache
def _sc_kernels():
    mesh = plsc.VectorSubcoreMesh(core_axis_name="c", subcore_axis_name="s")

    @functools.partial(
        pl.kernel,
        mesh=mesh,
        out_type=jax.ShapeDtypeStruct((NN, O), jnp.float32),
        scratch_types=[
            pltpu.VMEM((_GD, _C2), jnp.int32),
            pltpu.VMEM((_GD, _C2), jnp.int32),
            pltpu.VMEM((128,), jnp.float32),
            pltpu.VMEM((_DT,), jnp.float32),
            pltpu.VMEM((_DT,), jnp.float32),
            pltpu.VMEM((_DT, O), jnp.float32),
            pltpu.VMEM_SHARED((NN,), jnp.float32),
            pltpu.SemaphoreType.DMA,
            pltpu.SemaphoreType.DMA,
        ],
        compiler_params=pltpu.CompilerParams(use_tc_tiling_on_sc=False,
                                             needs_layout_passes=False),
    )
    def _sc_deg(ei_hbm, out_hbm, idx_a, idx_b, ones_v, dbuf, dsbuf, obuf,
                deg_sh, sem_a, sem_b):
        cid = lax.axis_index("c")
        sid = lax.axis_index("s")
        for j in range(128 // 16):
            ones_v[pl.ds(j * 16, 16)] = jnp.ones((16,), jnp.float32)

        def zb(j, carry):
            dbuf[pl.ds(pl.multiple_of(j * 16, 8), 16)] = (
                jnp.zeros((16,), jnp.float32))
            return carry

        lax.fori_loop(0, _DT // 16, zb, 0)

        @pl.when(sid < 10)
        def _():
            pltpu.sync_copy(dbuf, deg_sh.at[pl.ds(sid * _DT, _DT)])
            pltpu.sync_copy(dbuf,
                            deg_sh.at[pl.ds(NN // 2 + sid * _DT, _DT)])

        plsc.subcore_barrier()
        rbase = sid * (_E2 // 16)
        ngd = _E2 // 16 // _GD
        ones_s = ones_v.at[pl.ds(0, _C2)]

        def dfire(gg, idx_v, sem):
            pltpu.sync_copy(ei_hbm.at[1, pl.ds(rbase + gg * _GD, _GD)],
                            idx_v)
            for j in range(_GD):
                pltpu.async_copy(ones_s, deg_sh.at[idx_v.at[j]], sem,
                                 add=True)

        def ddrain(idx_v, sem):
            for j in range(_GD):
                pltpu.make_async_copy(ones_s, deg_sh.at[idx_v.at[j]],
                                      sem).wait()

        dfire(0, idx_a, sem_a)

        def body(it, carry):
            gg = it * 2
            dfire(gg + 1, idx_b, sem_b)
            ddrain(idx_a, sem_a)

            @pl.when(it < ngd // 2 - 1)
            def _():
                dfire(gg + 2, idx_a, sem_a)

            ddrain(idx_b, sem_b)
            return carry

        lax.fori_loop(0, ngd // 2, body, 0)
        plsc.subcore_barrier()

        @pl.when(sid < 10)
        def _():
            base = cid * (NN // 2) + sid * _DT
            pltpu.sync_copy(deg_sh.at[pl.ds(base, _DT)], dbuf)

            def newton(k16, carry):
                off = pl.multiple_of(k16 * 16, 8)
                x = dbuf[pl.ds(off, 16)] + 2.0
                bits = plsc.bitcast(x, jnp.int32)
                y = plsc.bitcast(0x5F3759DF - (bits >> 1), jnp.float32)
                hx = 0.5 * x
                y = y * (1.5 - hx * y * y)
                y = y * (1.5 - hx * y * y)
                y = y * (1.5 - hx * y * y)
                dsbuf[pl.ds(off, 16)] = y
                return carry

            lax.fori_loop(0, _DT // 16, newton, 0)

            def expand(r, carry):
                v16 = plsc.load_gather(dsbuf,
                                       [jnp.full((16,), r, jnp.int32)])
                obuf[r, pl.ds(0, 16)] = v16
                obuf[r, pl.ds(16, 16)] = v16
                return carry

            lax.fori_loop(0, _DT, expand, 0)
            pltpu.sync_copy(obuf, out_hbm.at[pl.ds(base, _DT)])

    @functools.partial(
        pl.kernel,
        mesh=mesh,
        out_type=jax.ShapeDtypeStruct((2, NN, O), jnp.float32),
        scratch_types=[
            pltpu.VMEM((2, _G, _C2), jnp.int32),
            pltpu.VMEM((2, _G, _C2), jnp.int32),
            pltpu.VMEM((_G, _C2, O), jnp.float32),
            pltpu.VMEM((_G, _C2, O), jnp.float32),
            pltpu.VMEM((_SB, O), jnp.float32),
            pltpu.VMEM_SHARED((NN, O), jnp.float32),
            pltpu.SemaphoreType.DMA,
            pltpu.SemaphoreType.DMA,
            pltpu.SemaphoreType.DMA,
            pltpu.SemaphoreType.DMA,
        ],
        compiler_params=pltpu.CompilerParams(use_tc_tiling_on_sc=False),
    )
    def _sc_scatter(ei_hbm, g_hbm, out_hbm,
                    idx_a, idx_b, rows_a, rows_b,
                    stage, acc_sh, gsem_a, gsem_b, ssem_a, ssem_b):
        cid = lax.axis_index("c")
        sid = lax.axis_index("s")
        wid = sid * 2 + cid

        def zb(i, carry):
            stage[i, pl.ds(0, 16)] = jnp.zeros((16,), jnp.float32)
            stage[i, pl.ds(16, 16)] = jnp.zeros((16,), jnp.float32)
            return carry

        lax.fori_loop(0, _SB, zb, 0)

        def zc(j, carry):
            pltpu.sync_copy(stage, acc_sh.at[pl.ds(sid * _WT + j * _SB,
                                                   _SB)])
            return carry

        lax.fori_loop(0, _WT // _SB, zc, 0)
        plsc.subcore_barrier()
        rbase = wid * _R2

        def drain_scat(idx_v, rows_v, ssem):
            for j in range(_G):
                pltpu.make_async_copy(rows_v.at[j],
                                      acc_sh.at[idx_v.at[1].at[j]],
                                      ssem).wait()

        def fire(gg, idx_v, rows_v, gsem, ssem):
            @pl.when(gg >= 2)
            def _():
                drain_scat(idx_v, rows_v, ssem)

            roff = rbase + gg * _G
            pltpu.sync_copy(ei_hbm.at[:, pl.ds(roff, _G)], idx_v)
            for j in range(_G):
                pltpu.async_copy(g_hbm.at[idx_v.at[0].at[j]],
                                 rows_v.at[j], gsem)

        def finish(idx_v, rows_v, gsem, ssem):
            for j in range(_G):
                pltpu.make_async_copy(g_hbm.at[idx_v.at[0].at[j]],
                                      rows_v.at[j], gsem).wait()
            for j in range(_G):
                pltpu.async_copy(rows_v.at[j], acc_sh.at[idx_v.at[1].at[j]],
                                 ssem, add=True)

        fire(0, idx_a, rows_a, gsem_a, ssem_a)

        def body(it, carry):
            gg = it * 2
            fire(gg + 1, idx_b, rows_b, gsem_b, ssem_b)
            finish(idx_a, rows_a, gsem_a, ssem_a)

            @pl.when(it < _NG // 2 - 1)
            def _():
                fire(gg + 2, idx_a, rows_a, gsem_a, ssem_a)

            finish(idx_b, rows_b, gsem_b, ssem_b)
            return carry

        lax.fori_loop(0, _NG // 2, body, 0)
        drain_scat(idx_a, rows_a, ssem_a)
        drain_scat(idx_b, rows_b, ssem_b)
        plsc.subcore_barrier()

        def wb(j, carry):
            off = sid * _WT + j * _SB
            pltpu.sync_copy(acc_sh.at[pl.ds(off, _SB)], stage)
            pltpu.sync_copy(stage, out_hbm.at[cid, pl.ds(off, _SB)])
            return carry

        lax.fori_loop(0, _WT // _SB, wb, 0)

    return _sc_deg, _sc_scatter


def kernel(input, edge_index, last_graph_weights, p, Wih, Whh, bih, bhh, Wg,
           bg):
    ei3 = edge_index.reshape(2, _E2, _C2)

    y2 = _proj(input, p.reshape(1, C), p.reshape(C, 1)).reshape(B, N)
    xs = _topk(input, y2)

    w = _gru(xs.reshape(B, GI), last_graph_weights,
             Wih.reshape(3, GH, GI), Whh.reshape(3, GH, GH),
             bih.reshape(3, 1, GH), bhh.reshape(3, 1, GH))

    sc_deg, sc_scatter = _sc_kernels()
    dis32 = sc_deg(ei3)

    h, g = _bmm(input, w.reshape(B, C, H), Wg, dis32)

    accp = sc_scatter(ei3, g)

    out = _fin(accp, dis32, h, bg.reshape(1, O))
    return out.reshape(B, N, O), w.reshape(B, C, H)
